# baseline (speedup 1.0000x reference)
import functools

import jax
import jax.numpy as jnp
from jax import lax
from jax.experimental import pallas as pl
from jax.experimental.pallas import tpu as pltpu
from jax.experimental.pallas import tpu_sc as plsc

NC = 2
NS = 16
NW = NC * NS
D = 8
B = 2000


def _edge_kernel_body(E, pos_hbm, edge_hbm, dist_hbm, vec_hbm,
                      row0, row1, col0, col1, prow0, prow1, pcol0, pcol1,
                      dist0, dist1, vx0, vx1, vy0, vy1, vz0, vz1,
                      isem0, isem1, gsem0, gsem1, osem0, osem1):
    epw = E // NW
    nch = epw // B
    wid = lax.axis_index("s") * NC + lax.axis_index("c")

    rows = (row0, row1)
    cols = (col0, col1)
    prows = (prow0, prow1)
    pcols = (pcol0, pcol1)
    dists = (dist0, dist1)
    vxs = (vx0, vx1)
    vys = (vy0, vy1)
    vzs = (vz0, vz1)
    isems = (isem0, isem1)
    gsems = (gsem0, gsem1)
    osems = (osem0, osem1)

    def idx_start(ci, s):
        base = wid * epw + ci * B
        pltpu.async_copy(edge_hbm.at[0, pl.ds(base, B)], rows[s], isems[s])
        pltpu.async_copy(edge_hbm.at[1, pl.ds(base, B)], cols[s], isems[s])

    def idx_wait(s):
        pltpu.make_async_copy(edge_hbm.at[0, pl.ds(0, B)], rows[s], isems[s]).wait()
        pltpu.make_async_copy(edge_hbm.at[1, pl.ds(0, B)], cols[s], isems[s]).wait()

    def g_start(s):
        pltpu.async_copy(pos_hbm.at[rows[s]], prows[s], gsems[s])
        pltpu.async_copy(pos_hbm.at[cols[s]], pcols[s], gsems[s])

    def g_wait(s):
        pltpu.make_async_copy(pos_hbm.at[rows[s]], prows[s], gsems[s]).wait()
        pltpu.make_async_copy(pos_hbm.at[cols[s]], pcols[s], gsems[s]).wait()

    def out_start(ci, s):
        base = wid * epw + ci * B
        pltpu.async_copy(dists[s], dist_hbm.at[pl.ds(base, B)], osems[s])
        pltpu.async_copy(vxs[s], vec_hbm.at[0, pl.ds(base, B)], osems[s])
        pltpu.async_copy(vys[s], vec_hbm.at[1, pl.ds(base, B)], osems[s])
        pltpu.async_copy(vzs[s], vec_hbm.at[2, pl.ds(base, B)], osems[s])

    def out_wait(s):
        pltpu.make_async_copy(dists[s], dist_hbm.at[pl.ds(0, B)], osems[s]).wait()
        pltpu.make_async_copy(vxs[s], vec_hbm.at[0, pl.ds(0, B)], osems[s]).wait()
        pltpu.make_async_copy(vys[s], vec_hbm.at[1, pl.ds(0, B)], osems[s]).wait()
        pltpu.make_async_copy(vzs[s], vec_hbm.at[2, pl.ds(0, B)], osems[s]).wait()

    def compute(s):
        rv, cv = rows[s], cols[s]
        pr, pc = prows[s], pcols[s]
        dv, xv, yv, zv = dists[s], vxs[s], vys[s], vzs[s]

        def lane_body(j, carry2):
            o = j * 16
            lid = o + lax.iota(jnp.int32, 16)
            k0 = jnp.zeros((16,), jnp.int32)
            k1 = jnp.full((16,), 1, jnp.int32)
            k2 = jnp.full((16,), 2, jnp.int32)
            rx = plsc.load_gather(pr, [lid, k0])
            ry = plsc.load_gather(pr, [lid, k1])
            rz = plsc.load_gather(pr, [lid, k2])
            cx = plsc.load_gather(pc, [lid, k0])
            cy = plsc.load_gather(pc, [lid, k1])
            cz = plsc.load_gather(pc, [lid, k2])
            dx = rx - cx
            dy = ry - cy
            dz = rz - cz
            sq = dx * dx + dy * dy + dz * dz
            r16 = rv[pl.ds(o, 16)]
            c16 = cv[pl.ds(o, 16)]
            sqs = jnp.where(r16 != c16, sq, 1.0)
            ibits = plsc.bitcast(sqs, jnp.int32)
            ibits = 0x5F3759DF - lax.shift_right_logical(ibits, 1)
            y = plsc.bitcast(ibits, jnp.float32)
            nh = sqs * -0.5
            y = y * (1.5 + nh * y * y)
            y = y * (1.5 + nh * y * y)
            dv[pl.ds(o, 16)] = sq * y
            xv[pl.ds(o, 16)] = dx * y
            yv[pl.ds(o, 16)] = dy * y
            zv[pl.ds(o, 16)] = dz * y
            return carry2

        lax.fori_loop(0, B // 16, lane_body, 0, unroll=5)

    idx_start(0, 0)
    idx_wait(0)
    g_start(0)
    idx_start(1, 1)

    @pl.loop(0, nch + 1, step=2)
    def _chunks(k):
        for s in (0, 1):
            ci = k + s

            @pl.when(ci < nch)
            def _step():
                @pl.when(ci + 1 < nch)
                def _prefetch_gather():
                    idx_wait(1 - s)
                    g_start(1 - s)

                g_wait(s)

                @pl.when(ci >= 2)
                def _drain_out():
                    out_wait(s)

                compute(s)
                out_start(ci, s)

                @pl.when(ci + 2 < nch)
                def _prefetch_idx():
                    idx_start(ci + 2, s)

    out_wait(1 - (nch - 1) % 2)
    out_wait((nch - 1) % 2)


def _edge_connect_sc(positions, edge_indices):
    E = edge_indices.shape[1]
    mesh = plsc.VectorSubcoreMesh(core_axis_name="c", subcore_axis_name="s",
                                  num_cores=NC, num_subcores=NS)
    body = functools.partial(_edge_kernel_body, E)
    return pl.kernel(
        body,
        out_type=[
            jax.ShapeDtypeStruct((E,), jnp.float32),
            jax.ShapeDtypeStruct((3, E), jnp.float32),
        ],
        mesh=mesh,
        compiler_params=pltpu.CompilerParams(needs_layout_passes=False,
                                             use_tc_tiling_on_sc=False),
        scratch_types=[
            pltpu.VMEM((B,), jnp.int32),
            pltpu.VMEM((B,), jnp.int32),
            pltpu.VMEM((B,), jnp.int32),
            pltpu.VMEM((B,), jnp.int32),
            pltpu.VMEM((B, D), jnp.float32),
            pltpu.VMEM((B, D), jnp.float32),
            pltpu.VMEM((B, D), jnp.float32),
            pltpu.VMEM((B, D), jnp.float32),
            pltpu.VMEM((B,), jnp.float32),
            pltpu.VMEM((B,), jnp.float32),
            pltpu.VMEM((B,), jnp.float32),
            pltpu.VMEM((B,), jnp.float32),
            pltpu.VMEM((B,), jnp.float32),
            pltpu.VMEM((B,), jnp.float32),
            pltpu.VMEM((B,), jnp.float32),
            pltpu.VMEM((B,), jnp.float32),
            pltpu.SemaphoreType.DMA,
            pltpu.SemaphoreType.DMA,
            pltpu.SemaphoreType.DMA,
            pltpu.SemaphoreType.DMA,
            pltpu.SemaphoreType.DMA,
            pltpu.SemaphoreType.DMA,
        ],
    )(positions, edge_indices)


def kernel(positions, batch, edge_indices):
    n = positions.shape[0]
    pos_pad = jnp.concatenate(
        [positions, jnp.zeros((n, D - 3), jnp.float32)], axis=1)
    dist, vec_t = _edge_connect_sc(pos_pad, edge_indices.astype(jnp.int32))
    vec = vec_t.T
    return (edge_indices, dist, vec)


# speedup vs baseline: 2.1161x; 2.1161x over previous
import functools

import jax
import jax.numpy as jnp
from jax import lax
from jax.experimental import pallas as pl
from jax.experimental.pallas import tpu as pltpu
from jax.experimental.pallas import tpu_sc as plsc

NC = 2
NS = 16
NW = NC * NS
D = 4
B = 2000


def _edge_kernel_body(E, pos_hbm, edge_hbm, dist_hbm, vx_hbm, vy_hbm, vz_hbm,
                      row0, row1, col0, col1, prow0, prow1, pcol0, pcol1,
                      dist0, dist1, vx0, vx1, vy0, vy1, vz0, vz1,
                      isem0, isem1, gsem0, gsem1, osem0, osem1):
    epw = E // NW
    nch = epw // B
    wid = lax.axis_index("s") * NC + lax.axis_index("c")

    rows = (row0, row1)
    cols = (col0, col1)
    prows = (prow0, prow1)
    pcols = (pcol0, pcol1)
    dists = (dist0, dist1)
    vxs = (vx0, vx1)
    vys = (vy0, vy1)
    vzs = (vz0, vz1)
    isems = (isem0, isem1)
    gsems = (gsem0, gsem1)
    osems = (osem0, osem1)

    def idx_start(ci, s):
        base = wid * epw + ci * B
        pltpu.async_copy(edge_hbm.at[0, pl.ds(base, B)], rows[s], isems[s])
        pltpu.async_copy(edge_hbm.at[1, pl.ds(base, B)], cols[s], isems[s])

    def idx_wait(s):
        pltpu.make_async_copy(edge_hbm.at[0, pl.ds(0, B)], rows[s], isems[s]).wait()
        pltpu.make_async_copy(edge_hbm.at[1, pl.ds(0, B)], cols[s], isems[s]).wait()

    def g_start(s):
        pltpu.async_copy(pos_hbm.at[rows[s]], prows[s], gsems[s])
        pltpu.async_copy(pos_hbm.at[cols[s]], pcols[s], gsems[s])

    def g_wait(s):
        pltpu.make_async_copy(pos_hbm.at[rows[s]], prows[s], gsems[s]).wait()
        pltpu.make_async_copy(pos_hbm.at[cols[s]], pcols[s], gsems[s]).wait()

    def out_start(ci, s):
        base = wid * epw + ci * B
        pltpu.async_copy(dists[s], dist_hbm.at[pl.ds(base, B)], osems[s])
        pltpu.async_copy(vxs[s], vx_hbm.at[pl.ds(base, B)], osems[s])
        pltpu.async_copy(vys[s], vy_hbm.at[pl.ds(base, B)], osems[s])
        pltpu.async_copy(vzs[s], vz_hbm.at[pl.ds(base, B)], osems[s])

    def out_wait(s):
        pltpu.make_async_copy(dists[s], dist_hbm.at[pl.ds(0, B)], osems[s]).wait()
        pltpu.make_async_copy(vxs[s], vx_hbm.at[pl.ds(0, B)], osems[s]).wait()
        pltpu.make_async_copy(vys[s], vy_hbm.at[pl.ds(0, B)], osems[s]).wait()
        pltpu.make_async_copy(vzs[s], vz_hbm.at[pl.ds(0, B)], osems[s]).wait()

    def compute(s):
        rv, cv = rows[s], cols[s]
        pr, pc = prows[s], pcols[s]
        dv, xv, yv, zv = dists[s], vxs[s], vys[s], vzs[s]

        def lane_body(j, carry2):
            o = j * 16
            lid = o + lax.iota(jnp.int32, 16)
            k0 = jnp.zeros((16,), jnp.int32)
            k1 = jnp.full((16,), 1, jnp.int32)
            k2 = jnp.full((16,), 2, jnp.int32)
            rx = plsc.load_gather(pr, [lid, k0])
            ry = plsc.load_gather(pr, [lid, k1])
            rz = plsc.load_gather(pr, [lid, k2])
            cx = plsc.load_gather(pc, [lid, k0])
            cy = plsc.load_gather(pc, [lid, k1])
            cz = plsc.load_gather(pc, [lid, k2])
            dx = rx - cx
            dy = ry - cy
            dz = rz - cz
            sq = dx * dx + dy * dy + dz * dz
            r16 = rv[pl.ds(o, 16)]
            c16 = cv[pl.ds(o, 16)]
            sqs = jnp.where(r16 != c16, sq, 1.0)
            ibits = plsc.bitcast(sqs, jnp.int32)
            ibits = 0x5F3759DF - lax.shift_right_logical(ibits, 1)
            y = plsc.bitcast(ibits, jnp.float32)
            nh = sqs * -0.5
            y = y * (1.5 + nh * y * y)
            y = y * (1.5 + nh * y * y)
            dv[pl.ds(o, 16)] = sq * y
            xv[pl.ds(o, 16)] = dx * y
            yv[pl.ds(o, 16)] = dy * y
            zv[pl.ds(o, 16)] = dz * y
            return carry2

        lax.fori_loop(0, B // 16, lane_body, 0, unroll=5)

    idx_start(0, 0)
    idx_wait(0)
    g_start(0)
    idx_start(1, 1)

    @pl.loop(0, nch + 1, step=2)
    def _chunks(k):
        for s in (0, 1):
            ci = k + s

            @pl.when(ci < nch)
            def _step():
                @pl.when(ci + 1 < nch)
                def _prefetch_gather():
                    idx_wait(1 - s)
                    g_start(1 - s)

                g_wait(s)

                @pl.when(ci >= 2)
                def _drain_out():
                    out_wait(s)

                compute(s)
                out_start(ci, s)

                @pl.when(ci + 2 < nch)
                def _prefetch_idx():
                    idx_start(ci + 2, s)

    out_wait(1 - (nch - 1) % 2)
    out_wait((nch - 1) % 2)


def _edge_connect_sc(positions, edge_indices):
    E = edge_indices.shape[1]
    mesh = plsc.VectorSubcoreMesh(core_axis_name="c", subcore_axis_name="s",
                                  num_cores=NC, num_subcores=NS)
    body = functools.partial(_edge_kernel_body, E)
    return pl.kernel(
        body,
        out_type=[
            jax.ShapeDtypeStruct((E,), jnp.float32),
            jax.ShapeDtypeStruct((E,), jnp.float32),
            jax.ShapeDtypeStruct((E,), jnp.float32),
            jax.ShapeDtypeStruct((E,), jnp.float32),
        ],
        mesh=mesh,
        compiler_params=pltpu.CompilerParams(needs_layout_passes=False,
                                             use_tc_tiling_on_sc=False),
        scratch_types=[
            pltpu.VMEM((B,), jnp.int32),
            pltpu.VMEM((B,), jnp.int32),
            pltpu.VMEM((B,), jnp.int32),
            pltpu.VMEM((B,), jnp.int32),
            pltpu.VMEM((B, D), jnp.float32),
            pltpu.VMEM((B, D), jnp.float32),
            pltpu.VMEM((B, D), jnp.float32),
            pltpu.VMEM((B, D), jnp.float32),
            pltpu.VMEM((B,), jnp.float32),
            pltpu.VMEM((B,), jnp.float32),
            pltpu.VMEM((B,), jnp.float32),
            pltpu.VMEM((B,), jnp.float32),
            pltpu.VMEM((B,), jnp.float32),
            pltpu.VMEM((B,), jnp.float32),
            pltpu.VMEM((B,), jnp.float32),
            pltpu.VMEM((B,), jnp.float32),
            pltpu.SemaphoreType.DMA,
            pltpu.SemaphoreType.DMA,
            pltpu.SemaphoreType.DMA,
            pltpu.SemaphoreType.DMA,
            pltpu.SemaphoreType.DMA,
            pltpu.SemaphoreType.DMA,
        ],
    )(positions, edge_indices)


def kernel(positions, batch, edge_indices):
    n = positions.shape[0]
    pos_pad = jnp.concatenate(
        [positions, jnp.zeros((n, D - 3), jnp.float32)], axis=1)
    dist, vx, vy, vz = _edge_connect_sc(pos_pad, edge_indices.astype(jnp.int32))
    vec = jnp.stack([vx, vy, vz], axis=1)
    return (edge_indices, dist, vec)


# speedup vs baseline: 2.7437x; 1.2966x over previous
import functools

import jax
import jax.numpy as jnp
from jax import lax
from jax.experimental import pallas as pl
from jax.experimental.pallas import tpu as pltpu
from jax.experimental.pallas import tpu_sc as plsc

NC = 2
NS = 16
NW = NC * NS
D = 8
CB = 16
B = CB * 128


def _edge_kernel_body(E, pos_hbm, edge_hbm, dist_hbm, vec_hbm,
                      row0, row1, col0, col1, prow0, prow1, pcol0, pcol1,
                      dist0, dist1, vout0, vout1,
                      isem0, isem1, gsem0, gsem1, osem0, osem1):
    nb_total = E // 128
    base_nb = nb_total // NW
    extra = nb_total - base_nb * NW
    wid = lax.axis_index("s") * NC + lax.axis_index("c")
    start = wid * base_nb + jnp.minimum(wid, extra)
    nb = base_nb + jnp.where(wid < extra, 1, 0)
    nch = (base_nb + CB - 1) // CB
    last_cb = start + nb - CB

    rows = (row0, row1)
    cols = (col0, col1)
    prows = (prow0, prow1)
    pcols = (pcol0, pcol1)
    dists = (dist0, dist1)
    vouts = (vout0, vout1)
    isems = (isem0, isem1)
    gsems = (gsem0, gsem1)
    osems = (osem0, osem1)

    def chunk_base(ci):
        return jnp.minimum(start + ci * CB, last_cb)

    def idx_start(ci, s):
        e0 = chunk_base(ci) * 128
        pltpu.async_copy(edge_hbm.at[0, pl.ds(e0, B)], rows[s], isems[s])
        pltpu.async_copy(edge_hbm.at[1, pl.ds(e0, B)], cols[s], isems[s])

    def idx_wait(s):
        pltpu.make_async_copy(edge_hbm.at[0, pl.ds(0, B)], rows[s], isems[s]).wait()
        pltpu.make_async_copy(edge_hbm.at[1, pl.ds(0, B)], cols[s], isems[s]).wait()

    def g_start(s):
        pltpu.async_copy(pos_hbm.at[rows[s]], prows[s], gsems[s])
        pltpu.async_copy(pos_hbm.at[cols[s]], pcols[s], gsems[s])

    def g_wait(s):
        pltpu.make_async_copy(pos_hbm.at[rows[s]], prows[s], gsems[s]).wait()
        pltpu.make_async_copy(pos_hbm.at[cols[s]], pcols[s], gsems[s]).wait()

    def out_start(ci, s):
        cb = chunk_base(ci)
        pltpu.async_copy(dists[s], dist_hbm.at[pl.ds(cb * 128, B)], osems[s])
        pltpu.async_copy(vouts[s], vec_hbm.at[pl.ds(cb, CB)], osems[s])

    def out_wait(s):
        pltpu.make_async_copy(dists[s], dist_hbm.at[pl.ds(0, B)], osems[s]).wait()
        pltpu.make_async_copy(vouts[s], vec_hbm.at[pl.ds(0, CB)], osems[s]).wait()

    def compute(s):
        rv, cv = rows[s], cols[s]
        pr, pc = prows[s], pcols[s]
        dv, vo = dists[s], vouts[s]

        def block_body(bi, carry):
            for jj in range(8):
                o = bi * 128 + jj * 16
                ci16 = jj * 16
                lid = o + lax.iota(jnp.int32, 16)
                k0 = jnp.zeros((16,), jnp.int32)
                k1 = jnp.full((16,), 1, jnp.int32)
                k2 = jnp.full((16,), 2, jnp.int32)
                rx = plsc.load_gather(pr, [lid, k0])
                ry = plsc.load_gather(pr, [lid, k1])
                rz = plsc.load_gather(pr, [lid, k2])
                cx = plsc.load_gather(pc, [lid, k0])
                cy = plsc.load_gather(pc, [lid, k1])
                cz = plsc.load_gather(pc, [lid, k2])
                dx = rx - cx
                dy = ry - cy
                dz = rz - cz
                sq = dx * dx + dy * dy + dz * dz
                r16 = rv[pl.ds(o, 16)]
                c16 = cv[pl.ds(o, 16)]
                sqs = jnp.where(r16 != c16, sq, 1.0)
                ibits = plsc.bitcast(sqs, jnp.int32)
                ibits = 0x5F3759DF - lax.shift_right_logical(ibits, 1)
                y = plsc.bitcast(ibits, jnp.float32)
                nh = sqs * -0.5
                y = y * (1.5 + nh * y * y)
                y = y * (1.5 + nh * y * y)
                dv[pl.ds(o, 16)] = sq * y
                vo[bi, 0, pl.ds(ci16, 16)] = dx * y
                vo[bi, 1, pl.ds(ci16, 16)] = dy * y
                vo[bi, 2, pl.ds(ci16, 16)] = dz * y
            return carry

        lax.fori_loop(0, CB, block_body, 0, unroll=False)

    idx_start(0, 0)
    idx_wait(0)
    g_start(0)
    idx_start(1, 1)

    @pl.loop(0, nch + 1, step=2)
    def _chunks(k):
        for s in (0, 1):
            ci = k + s

            @pl.when(ci < nch)
            def _step():
                @pl.when(ci + 1 < nch)
                def _prefetch_gather():
                    idx_wait(1 - s)
                    g_start(1 - s)

                g_wait(s)

                @pl.when(ci >= 2)
                def _drain_out():
                    out_wait(s)

                compute(s)
                out_start(ci, s)

                @pl.when(ci + 2 < nch)
                def _prefetch_idx():
                    idx_start(ci + 2, s)

    out_wait(1 - (nch - 1) % 2)
    out_wait((nch - 1) % 2)


def _edge_connect_sc(positions, edge_indices):
    E = edge_indices.shape[1]
    mesh = plsc.VectorSubcoreMesh(core_axis_name="c", subcore_axis_name="s",
                                  num_cores=NC, num_subcores=NS)
    body = functools.partial(_edge_kernel_body, E)
    return pl.kernel(
        body,
        out_type=[
            jax.ShapeDtypeStruct((E,), jnp.float32),
            jax.ShapeDtypeStruct((E // 128, 4, 128), jnp.float32),
        ],
        mesh=mesh,
        compiler_params=pltpu.CompilerParams(needs_layout_passes=False,
                                             use_tc_tiling_on_sc=False),
        scratch_types=[
            pltpu.VMEM((B,), jnp.int32),
            pltpu.VMEM((B,), jnp.int32),
            pltpu.VMEM((B,), jnp.int32),
            pltpu.VMEM((B,), jnp.int32),
            pltpu.VMEM((B, D), jnp.float32),
            pltpu.VMEM((B, D), jnp.float32),
            pltpu.VMEM((B, D), jnp.float32),
            pltpu.VMEM((B, D), jnp.float32),
            pltpu.VMEM((B,), jnp.float32),
            pltpu.VMEM((B,), jnp.float32),
            pltpu.VMEM((CB, 4, 128), jnp.float32),
            pltpu.VMEM((CB, 4, 128), jnp.float32),
            pltpu.SemaphoreType.DMA,
            pltpu.SemaphoreType.DMA,
            pltpu.SemaphoreType.DMA,
            pltpu.SemaphoreType.DMA,
            pltpu.SemaphoreType.DMA,
            pltpu.SemaphoreType.DMA,
        ],
    )(positions, edge_indices)


def kernel(positions, batch, edge_indices):
    n = positions.shape[0]
    e = edge_indices.shape[1]
    pos_pad = jnp.concatenate(
        [positions, jnp.zeros((n, D - 3), jnp.float32)], axis=1)
    dist, vec3d = _edge_connect_sc(pos_pad, edge_indices.astype(jnp.int32))
    vec = vec3d[:, :3, :].transpose(0, 2, 1).reshape(e, 3)
    return (edge_indices, dist, vec)
